# baseline (speedup 1.0000x reference)
import functools

import jax
import jax.numpy as jnp
from jax import lax
from jax.experimental import pallas as pl
from jax.experimental.pallas import tpu as pltpu
from jax.experimental.pallas import tpu_sc as plsc

_NW = 32
_B = 128
_L = 16
_H = 128


def _tc_tables(nf, Wn1, bn1, Wn2, bn2, We1a, We1b, be1):
    n = nf.shape[0]
    d = nf.shape[1]
    bn = 400
    assert n % bn == 0

    def body(nf_ref, wn1_ref, bn1_ref, wn2_ref, bn2_ref, we1a_ref, we1b_ref,
             be1_ref, ns_ref, t1_ref, t2_ref):
        x = nf_ref[...]
        h = jnp.maximum(
            jnp.dot(x, wn1_ref[...], preferred_element_type=jnp.float32)
            + bn1_ref[...], 0.0)
        ns_ref[...] = (jnp.sum(h * wn2_ref[...], axis=1, keepdims=True)
                       + bn2_ref[...])
        t1_ref[...] = (jnp.dot(x, we1a_ref[...],
                               preferred_element_type=jnp.float32)
                       + be1_ref[...])
        t2_ref[...] = jnp.dot(x, we1b_ref[...],
                              preferred_element_type=jnp.float32)

    return pl.pallas_call(
        body,
        grid=(n // bn,),
        in_specs=[
            pl.BlockSpec((bn, d), lambda i: (i, 0)),
            pl.BlockSpec((d, _H), lambda i: (0, 0)),
            pl.BlockSpec((1, _H), lambda i: (0, 0)),
            pl.BlockSpec((1, _H), lambda i: (0, 0)),
            pl.BlockSpec((1, 1), lambda i: (0, 0)),
            pl.BlockSpec((d, _H), lambda i: (0, 0)),
            pl.BlockSpec((d, _H), lambda i: (0, 0)),
            pl.BlockSpec((1, _H), lambda i: (0, 0)),
        ],
        out_specs=[
            pl.BlockSpec((bn, 1), lambda i: (i, 0)),
            pl.BlockSpec((bn, _H), lambda i: (i, 0)),
            pl.BlockSpec((bn, _H), lambda i: (i, 0)),
        ],
        out_shape=[
            jax.ShapeDtypeStruct((n, 1), jnp.float32),
            jax.ShapeDtypeStruct((n, _H), jnp.float32),
            jax.ShapeDtypeStruct((n, _H), jnp.float32),
        ],
    )(nf, Wn1, bn1, Wn2, bn2, We1a, We1b, be1)


def _sc_edge_partials(t1, t2, esrc, edst, w2):
    epad = esrc.shape[0]
    nc = epad // (_NW * _B)
    assert nc % 2 == 0
    mesh = plsc.VectorSubcoreMesh(core_axis_name="c", subcore_axis_name="s")

    @functools.partial(
        pl.kernel,
        mesh=mesh,
        out_type=jax.ShapeDtypeStruct((epad * _L,), jnp.float32),
        scratch_types=[
            pltpu.VMEM((2, _B), jnp.int32),
            pltpu.VMEM((2, _B), jnp.int32),
            pltpu.VMEM((2, _B, _H), jnp.float32),
            pltpu.VMEM((2, _B, _H), jnp.float32),
            pltpu.VMEM((2, _B * _L), jnp.float32),
            pltpu.VMEM((_H,), jnp.float32),
            pltpu.SemaphoreType.DMA,
            pltpu.SemaphoreType.DMA,
            pltpu.SemaphoreType.DMA,
            pltpu.SemaphoreType.DMA,
            pltpu.SemaphoreType.DMA,
            pltpu.SemaphoreType.DMA,
        ],
    )
    def k(t1_hbm, t2_hbm, esrc_hbm, edst_hbm, w2_hbm, out_hbm,
          sidx2, didx2, srows2, drows2, outv2, w2v,
          sg0, sg1, si0, si1, so0, so1):
        sem_g, sem_i, sem_o = [sg0, sg1], [si0, si1], [so0, so1]
        wid = lax.axis_index("s") * 2 + lax.axis_index("c")
        pltpu.sync_copy(w2_hbm, w2v)
        w2c = [w2v[pl.ds(_L * j, _L)] for j in range(_H // _L)]

        def base_of(c):
            return (c * _NW + wid) * _B

        def issue_idx(c, b):
            base = base_of(c)
            pltpu.async_copy(esrc_hbm.at[pl.ds(base, _B)], sidx2.at[b],
                             sem_i[b])
            pltpu.async_copy(edst_hbm.at[pl.ds(base, _B)], didx2.at[b],
                             sem_i[b])

        def wait_idx(b):
            pltpu.make_async_copy(esrc_hbm.at[pl.ds(0, _B)], sidx2.at[b],
                                  sem_i[b]).wait()
            pltpu.make_async_copy(edst_hbm.at[pl.ds(0, _B)], didx2.at[b],
                                  sem_i[b]).wait()

        def issue_gather(b):
            pltpu.async_copy(t1_hbm.at[sidx2.at[b]], srows2.at[b], sem_g[b])
            pltpu.async_copy(t2_hbm.at[didx2.at[b]], drows2.at[b], sem_g[b])

        def wait_gather(b):
            pltpu.make_async_copy(t1_hbm.at[sidx2.at[b]], srows2.at[b],
                                  sem_g[b]).wait()
            pltpu.make_async_copy(t2_hbm.at[didx2.at[b]], drows2.at[b],
                                  sem_g[b]).wait()

        def issue_out(c, b):
            pltpu.async_copy(outv2.at[b],
                             out_hbm.at[pl.ds(base_of(c) * _L, _B * _L)],
                             sem_o[b])

        def wait_out(b):
            pltpu.make_async_copy(outv2.at[b],
                                  out_hbm.at[pl.ds(0, _B * _L)],
                                  sem_o[b]).wait()

        def compute(c, b):
            srows, drows = srows2.at[b], drows2.at[b]
            outv = outv2.at[b]

            def edge_body(e, cc):
                parts = []
                for j in range(_H // _L):
                    sl = pl.ds(_L * j, _L)
                    u = jnp.maximum(srows[e, sl] + drows[e, sl], 0.0)
                    parts.append(u * w2c[j])
                while len(parts) > 1:
                    parts = [a + b_ for a, b_ in
                             zip(parts[::2], parts[1::2])]
                outv[pl.ds(e * _L, _L)] = parts[0]
                return cc

            lax.fori_loop(0, _B, edge_body, 0, unroll=4)

        issue_idx(0, 0)
        wait_idx(0)
        issue_gather(0)
        issue_idx(1, 1)

        def pair_body(kk, carry):
            for b in (0, 1):
                c = 2 * kk + b
                wait_idx(b ^ 1)
                issue_gather(b ^ 1)
                wait_gather(b)
                issue_idx(jnp.minimum(c + 2, nc - 1), b)
                @pl.when(c >= 2)
                def _():
                    wait_out(b)
                compute(c, b)
                issue_out(c, b)
            return carry

        lax.fori_loop(0, nc // 2, pair_body, 0)
        wait_gather(0)
        wait_idx(1)
        wait_out(0)
        wait_out(1)

    return k(t1, t2, esrc, edst, w2)


def _tc_finalize(partials, seg, b2):
    r = partials.shape[0]
    br = 512
    assert r % br == 0

    def body(p_ref, s_ref, b2_ref, o_ref):
        o_ref[...] = (jnp.dot(p_ref[...], s_ref[...],
                              preferred_element_type=jnp.float32)
                      + b2_ref[...])

    return pl.pallas_call(
        body,
        grid=(r // br,),
        in_specs=[
            pl.BlockSpec((br, 128), lambda i: (i, 0)),
            pl.BlockSpec((128, 8), lambda i: (0, 0)),
            pl.BlockSpec((1, 1), lambda i: (0, 0)),
        ],
        out_specs=pl.BlockSpec((br, 8), lambda i: (i, 0)),
        out_shape=jax.ShapeDtypeStruct((r, 8), jnp.float32),
    )(partials, seg, b2)


def kernel(node_feats, node_xy, node_adj_ids, edge_ids, Wn1, bn1, Wn2, bn2,
           We1, be1, We2, be2):
    d = node_feats.shape[1]
    e = edge_ids.shape[1]

    node_scores, t1, t2 = _tc_tables(
        node_feats, Wn1, bn1.reshape(1, -1), Wn2.reshape(1, -1),
        bn2.reshape(1, 1), We1[:d], We1[d:], be1.reshape(1, -1))

    epad = -(-e // (2 * _NW * _B)) * (2 * _NW * _B)
    esrc = jnp.pad(edge_ids[0], (0, epad - e))
    edst = jnp.pad(edge_ids[1], (0, epad - e))
    w2 = We2.reshape(-1)

    partials = _sc_edge_partials(t1, t2, esrc, edst, w2)
    seg = (jnp.arange(128)[:, None] // _L
           == jnp.arange(8)[None, :]).astype(jnp.float32)
    sums = _tc_finalize(partials.reshape(epad * _L // 128, 128), seg,
                        be2.reshape(1, 1))
    edge_scores = sums.reshape(epad, 1)[:e]
    return (node_scores, edge_scores)


# speedup vs baseline: 1.1203x; 1.1203x over previous
import functools

import jax
import jax.numpy as jnp
from jax import lax
from jax.experimental import pallas as pl
from jax.experimental.pallas import tpu as pltpu
from jax.experimental.pallas import tpu_sc as plsc

_NW = 32
_B = 128
_L = 16
_H = 128


def _tc_tables(nf, Wn1, bn1, Wn2, bn2, We1a, We1b, be1):
    n = nf.shape[0]
    d = nf.shape[1]
    bn = 400
    assert n % bn == 0

    def body(nf_ref, wn1_ref, bn1_ref, wn2_ref, bn2_ref, we1a_ref, we1b_ref,
             be1_ref, ns_ref, t1_ref, t2_ref):
        x = nf_ref[...]
        h = jnp.maximum(
            jnp.dot(x, wn1_ref[...], preferred_element_type=jnp.float32)
            + bn1_ref[...], 0.0)
        ns_ref[...] = (jnp.sum(h * wn2_ref[...], axis=1, keepdims=True)
                       + bn2_ref[...])
        t1_ref[...] = (jnp.dot(x, we1a_ref[...],
                               preferred_element_type=jnp.float32)
                       + be1_ref[...])
        t2_ref[...] = jnp.dot(x, we1b_ref[...],
                              preferred_element_type=jnp.float32)

    return pl.pallas_call(
        body,
        grid=(n // bn,),
        in_specs=[
            pl.BlockSpec((bn, d), lambda i: (i, 0)),
            pl.BlockSpec((d, _H), lambda i: (0, 0)),
            pl.BlockSpec((1, _H), lambda i: (0, 0)),
            pl.BlockSpec((1, _H), lambda i: (0, 0)),
            pl.BlockSpec((1, 1), lambda i: (0, 0)),
            pl.BlockSpec((d, _H), lambda i: (0, 0)),
            pl.BlockSpec((d, _H), lambda i: (0, 0)),
            pl.BlockSpec((1, _H), lambda i: (0, 0)),
        ],
        out_specs=[
            pl.BlockSpec((bn, 1), lambda i: (i, 0)),
            pl.BlockSpec((bn, _H), lambda i: (i, 0)),
            pl.BlockSpec((bn, _H), lambda i: (i, 0)),
        ],
        out_shape=[
            jax.ShapeDtypeStruct((n, 1), jnp.float32),
            jax.ShapeDtypeStruct((n, _H), jnp.float32),
            jax.ShapeDtypeStruct((n, _H), jnp.float32),
        ],
    )(nf, Wn1, bn1, Wn2, bn2, We1a, We1b, be1)


def _sc_edge_partials(t1, t2, esrc, edst, w2):
    epad = esrc.shape[0]
    nc = epad // (_NW * _B)
    assert nc % 2 == 0
    mesh = plsc.VectorSubcoreMesh(core_axis_name="c", subcore_axis_name="s")

    @functools.partial(
        pl.kernel,
        mesh=mesh,
        out_type=jax.ShapeDtypeStruct((epad * _L,), jnp.float32),
        scratch_types=[
            pltpu.VMEM((2, _B), jnp.int32),
            pltpu.VMEM((2, _B), jnp.int32),
            pltpu.VMEM((2, _B, _H), jnp.float32),
            pltpu.VMEM((2, _B, _H), jnp.float32),
            pltpu.VMEM((2, _B * _L), jnp.float32),
            pltpu.VMEM((_H,), jnp.float32),
            pltpu.SemaphoreType.DMA,
            pltpu.SemaphoreType.DMA,
            pltpu.SemaphoreType.DMA,
            pltpu.SemaphoreType.DMA,
            pltpu.SemaphoreType.DMA,
            pltpu.SemaphoreType.DMA,
        ],
    )
    def k(t1_hbm, t2_hbm, esrc_hbm, edst_hbm, w2_hbm, out_hbm,
          sidx2, didx2, srows2, drows2, outv2, w2v,
          sg0, sg1, si0, si1, so0, so1):
        sem_g, sem_i, sem_o = [sg0, sg1], [si0, si1], [so0, so1]
        wid = lax.axis_index("s") * 2 + lax.axis_index("c")
        pltpu.sync_copy(w2_hbm, w2v)
        w2c = [w2v[pl.ds(_L * j, _L)] for j in range(_H // _L)]

        def base_of(c):
            return (c * _NW + wid) * _B

        def issue_idx(c, b):
            base = base_of(c)
            pltpu.async_copy(esrc_hbm.at[pl.ds(base, _B)], sidx2.at[b],
                             sem_i[b])
            pltpu.async_copy(edst_hbm.at[pl.ds(base, _B)], didx2.at[b],
                             sem_i[b])

        def wait_idx(b):
            pltpu.make_async_copy(esrc_hbm.at[pl.ds(0, _B)], sidx2.at[b],
                                  sem_i[b]).wait()
            pltpu.make_async_copy(edst_hbm.at[pl.ds(0, _B)], didx2.at[b],
                                  sem_i[b]).wait()

        def issue_gather(b):
            pltpu.async_copy(t1_hbm.at[sidx2.at[b]], srows2.at[b], sem_g[b])
            pltpu.async_copy(t2_hbm.at[didx2.at[b]], drows2.at[b], sem_g[b])

        def wait_gather(b):
            pltpu.make_async_copy(t1_hbm.at[sidx2.at[b]], srows2.at[b],
                                  sem_g[b]).wait()
            pltpu.make_async_copy(t2_hbm.at[didx2.at[b]], drows2.at[b],
                                  sem_g[b]).wait()

        def issue_out(c, b):
            pltpu.async_copy(outv2.at[b],
                             out_hbm.at[pl.ds(base_of(c) * _L, _B * _L)],
                             sem_o[b])

        def wait_out(b):
            pltpu.make_async_copy(outv2.at[b],
                                  out_hbm.at[pl.ds(0, _B * _L)],
                                  sem_o[b]).wait()

        def compute(c, b):
            srows, drows = srows2.at[b], drows2.at[b]
            outv = outv2.at[b]

            @plsc.parallel_loop(0, _B, 1, unroll=4)
            def edge_body(e):
                parts = []
                for j in range(_H // _L):
                    sl = pl.ds(_L * j, _L)
                    u = jnp.maximum(srows[e, sl] + drows[e, sl], 0.0)
                    parts.append(u * w2c[j])
                while len(parts) > 1:
                    parts = [a + b_ for a, b_ in
                             zip(parts[::2], parts[1::2])]
                outv[pl.ds(e * _L, _L)] = parts[0]

        issue_idx(0, 0)
        wait_idx(0)
        issue_gather(0)
        issue_idx(1, 1)

        def pair_body(kk, carry):
            for b in (0, 1):
                c = 2 * kk + b
                wait_idx(b ^ 1)
                issue_gather(b ^ 1)
                wait_gather(b)
                issue_idx(jnp.minimum(c + 2, nc - 1), b)
                @pl.when(c >= 2)
                def _():
                    wait_out(b)
                compute(c, b)
                issue_out(c, b)
            return carry

        lax.fori_loop(0, nc // 2, pair_body, 0)
        wait_gather(0)
        wait_idx(1)
        wait_out(0)
        wait_out(1)

    return k(t1, t2, esrc, edst, w2)


def _tc_finalize(partials, seg, b2):
    r = partials.shape[0]
    br = 512
    assert r % br == 0

    def body(p_ref, s_ref, b2_ref, o_ref):
        o_ref[...] = (jnp.dot(p_ref[...], s_ref[...],
                              preferred_element_type=jnp.float32)
                      + b2_ref[...])

    return pl.pallas_call(
        body,
        grid=(r // br,),
        in_specs=[
            pl.BlockSpec((br, 128), lambda i: (i, 0)),
            pl.BlockSpec((128, 8), lambda i: (0, 0)),
            pl.BlockSpec((1, 1), lambda i: (0, 0)),
        ],
        out_specs=pl.BlockSpec((br, 8), lambda i: (i, 0)),
        out_shape=jax.ShapeDtypeStruct((r, 8), jnp.float32),
    )(partials, seg, b2)


def kernel(node_feats, node_xy, node_adj_ids, edge_ids, Wn1, bn1, Wn2, bn2,
           We1, be1, We2, be2):
    d = node_feats.shape[1]
    e = edge_ids.shape[1]

    node_scores, t1, t2 = _tc_tables(
        node_feats, Wn1, bn1.reshape(1, -1), Wn2.reshape(1, -1),
        bn2.reshape(1, 1), We1[:d], We1[d:], be1.reshape(1, -1))

    epad = -(-e // (2 * _NW * _B)) * (2 * _NW * _B)
    esrc = jnp.pad(edge_ids[0], (0, epad - e))
    edst = jnp.pad(edge_ids[1], (0, epad - e))
    w2 = We2.reshape(-1)

    partials = _sc_edge_partials(t1, t2, esrc, edst, w2)
    seg = (jnp.arange(128)[:, None] // _L
           == jnp.arange(8)[None, :]).astype(jnp.float32)
    sums = _tc_finalize(partials.reshape(epad * _L // 128, 128), seg,
                        be2.reshape(1, 1))
    edge_scores = sums.reshape(epad, 1)[:e]
    return (node_scores, edge_scores)


# speedup vs baseline: 1.6838x; 1.5030x over previous
import functools

import jax
import jax.numpy as jnp
from jax import lax
from jax.experimental import pallas as pl
from jax.experimental.pallas import tpu as pltpu
from jax.experimental.pallas import tpu_sc as plsc

_NW = 32
_B = 128
_L = 16
_H = 128
_QMAX = 32000.0
_RB = 64

_SC_PARAMS = pltpu.CompilerParams(use_tc_tiling_on_sc=False)


def _tc_tables(nf, Wn1, bn1, Wn2, bn2, We1a, We1b, be1):
    n = nf.shape[0]
    d = nf.shape[1]
    bn = 512
    assert n % bn == 0

    def body(nf_ref, wn1_ref, bn1_ref, wn2_ref, bn2_ref, we1a_ref, we1b_ref,
             be1_ref, ns_ref, t1_ref, t2_ref, am_ref):
        i = pl.program_id(0)
        x = nf_ref[...]
        h = jnp.maximum(
            jnp.dot(x, wn1_ref[...], preferred_element_type=jnp.float32)
            + bn1_ref[...], 0.0)
        ns_ref[...] = (jnp.sum(h * wn2_ref[...], axis=1, keepdims=True)
                       + bn2_ref[...])
        t1 = (jnp.dot(x, we1a_ref[...], preferred_element_type=jnp.float32)
              + be1_ref[...])
        t2 = jnp.dot(x, we1b_ref[...], preferred_element_type=jnp.float32)
        t1_ref[...] = t1
        t2_ref[...] = t2
        bm = jnp.maximum(jnp.max(jnp.abs(t1)), jnp.max(jnp.abs(t2)))

        @pl.when(i == 0)
        def _():
            am_ref[...] = jnp.full((1, _H), 1e-30, jnp.float32)

        am_ref[...] = jnp.maximum(am_ref[...], bm)

    return pl.pallas_call(
        body,
        grid=(n // bn,),
        in_specs=[
            pl.BlockSpec((bn, d), lambda i: (i, 0)),
            pl.BlockSpec((d, _H), lambda i: (0, 0)),
            pl.BlockSpec((1, _H), lambda i: (0, 0)),
            pl.BlockSpec((1, _H), lambda i: (0, 0)),
            pl.BlockSpec((1, 1), lambda i: (0, 0)),
            pl.BlockSpec((d, _H), lambda i: (0, 0)),
            pl.BlockSpec((d, _H), lambda i: (0, 0)),
            pl.BlockSpec((1, _H), lambda i: (0, 0)),
        ],
        out_specs=[
            pl.BlockSpec((bn, 1), lambda i: (i, 0)),
            pl.BlockSpec((bn, _H), lambda i: (i, 0)),
            pl.BlockSpec((bn, _H), lambda i: (i, 0)),
            pl.BlockSpec((1, _H), lambda i: (0, 0)),
        ],
        out_shape=[
            jax.ShapeDtypeStruct((n, 1), jnp.float32),
            jax.ShapeDtypeStruct((n, _H), jnp.float32),
            jax.ShapeDtypeStruct((n, _H), jnp.float32),
            jax.ShapeDtypeStruct((1, _H), jnp.float32),
        ],
    )(nf, Wn1, bn1, Wn2, bn2, We1a, We1b, be1)


def _sc_pack(t1, t2, amax):
    n = t1.shape[0]
    rows_w = n // _NW
    assert rows_w % _RB == 0
    mesh = plsc.VectorSubcoreMesh(core_axis_name="c", subcore_axis_name="s")

    @functools.partial(
        pl.kernel,
        mesh=mesh,
        compiler_params=_SC_PARAMS,
        out_type=[
            jax.ShapeDtypeStruct((n, _H // 2), jnp.int32),
            jax.ShapeDtypeStruct((n, _H // 2), jnp.int32),
        ],
        scratch_types=[
            pltpu.VMEM((_RB, _H), jnp.float32),
            pltpu.VMEM((_RB, _H), jnp.float32),
            pltpu.VMEM((_RB, _H // 2), jnp.int32),
            pltpu.VMEM((_RB, _H // 2), jnp.int32),
            pltpu.VMEM((_H,), jnp.float32),
        ],
    )
    def k(t1_hbm, t2_hbm, am_hbm, p1_hbm, p2_hbm, t1v, t2v, p1v, p2v, amv):
        wid = lax.axis_index("s") * 2 + lax.axis_index("c")
        pltpu.sync_copy(am_hbm.at[0], amv)
        inv = _QMAX / amv[pl.ds(0, _L)]

        def chunk_body(c, carry):
            r0 = wid * rows_w + c * _RB
            pltpu.sync_copy(t1_hbm.at[pl.ds(r0, _RB)], t1v)
            pltpu.sync_copy(t2_hbm.at[pl.ds(r0, _RB)], t2v)

            @plsc.parallel_loop(0, _RB, 1, unroll=2)
            def row_body(r):
                for tv, pv in ((t1v, p1v), (t2v, p2v)):
                    for j in range(_H // (2 * _L)):
                        lo = tv[r, pl.ds(_L * j, _L)] * inv
                        hi = tv[r, pl.ds(_H // 2 + _L * j, _L)] * inv
                        qlo = (lo + jnp.where(lo >= 0, 0.5, -0.5)
                               ).astype(jnp.int32)
                        qhi = (hi + jnp.where(hi >= 0, 0.5, -0.5)
                               ).astype(jnp.int32)
                        pv[r, pl.ds(_L * j, _L)] = (
                            (qlo & 0xFFFF) | (qhi << 16))

            pltpu.sync_copy(p1v, p1_hbm.at[pl.ds(r0, _RB)])
            pltpu.sync_copy(p2v, p2_hbm.at[pl.ds(r0, _RB)])
            return carry

        lax.fori_loop(0, rows_w // _RB, chunk_body, 0)

    return k(t1, t2, amax)


def _sc_edge_partials(p1, p2, esrc, edst, w2, amax):
    epad = esrc.shape[0]
    nc = epad // (_NW * _B)
    mesh = plsc.VectorSubcoreMesh(core_axis_name="c", subcore_axis_name="s")

    @functools.partial(
        pl.kernel,
        mesh=mesh,
        compiler_params=_SC_PARAMS,
        out_type=jax.ShapeDtypeStruct((epad * _L,), jnp.float32),
        scratch_types=[
            pltpu.VMEM((_B,), jnp.int32),
            pltpu.VMEM((_B,), jnp.int32),
            pltpu.VMEM((_B, _H // 2), jnp.int32),
            pltpu.VMEM((_B, _H // 2), jnp.int32),
            pltpu.VMEM((_B * _L,), jnp.float32),
            pltpu.VMEM((_H,), jnp.float32),
            pltpu.VMEM((_H,), jnp.float32),
            pltpu.SemaphoreType.DMA,
            pltpu.SemaphoreType.DMA,
        ],
    )
    def k(p1_hbm, p2_hbm, esrc_hbm, edst_hbm, w2_hbm, am_hbm, out_hbm,
          sidx, didx, srows, drows, outv, w2v, amv, sem1, sem2):
        wid = lax.axis_index("s") * 2 + lax.axis_index("c")
        pltpu.sync_copy(w2_hbm, w2v)
        pltpu.sync_copy(am_hbm.at[0], amv)
        step = amv[pl.ds(0, _L)] * (1.0 / _QMAX)
        w2c = [w2v[pl.ds(_L * j, _L)] * step for j in range(_H // _L)]
        nwc = _H // (2 * _L)

        def chunk_body(i, carry):
            base = (i * _NW + wid) * _B
            pltpu.sync_copy(esrc_hbm.at[pl.ds(base, _B)], sidx)
            pltpu.sync_copy(edst_hbm.at[pl.ds(base, _B)], didx)
            cp1 = pltpu.async_copy(p1_hbm.at[sidx], srows, sem1)
            cp2 = pltpu.async_copy(p2_hbm.at[didx], drows, sem2)
            cp1.wait()
            cp2.wait()

            @plsc.parallel_loop(0, _B, 1, unroll=4)
            def edge_body(e):
                parts = []
                for j in range(nwc):
                    sl = pl.ds(_L * j, _L)
                    sw = srows[e, sl]
                    dw = drows[e, sl]
                    u_lo = ((sw << 16) >> 16) + ((dw << 16) >> 16)
                    u_hi = (sw >> 16) + (dw >> 16)
                    f_lo = jnp.maximum(u_lo, 0).astype(jnp.float32)
                    f_hi = jnp.maximum(u_hi, 0).astype(jnp.float32)
                    parts.append(f_lo * w2c[j])
                    parts.append(f_hi * w2c[j + nwc])
                while len(parts) > 1:
                    parts = [a + b for a, b in zip(parts[::2], parts[1::2])]
                outv[pl.ds(e * _L, _L)] = parts[0]

            pltpu.sync_copy(outv, out_hbm.at[pl.ds(base * _L, _B * _L)])
            return carry

        lax.fori_loop(0, nc, chunk_body, 0)

    return k(p1, p2, esrc, edst, w2, amax)


def _tc_finalize(partials, seg, b2):
    r = partials.shape[0]
    br = 512
    assert r % br == 0

    def body(p_ref, s_ref, b2_ref, o_ref):
        o_ref[...] = (jnp.dot(p_ref[...], s_ref[...],
                              preferred_element_type=jnp.float32)
                      + b2_ref[...])

    return pl.pallas_call(
        body,
        grid=(r // br,),
        in_specs=[
            pl.BlockSpec((br, 128), lambda i: (i, 0)),
            pl.BlockSpec((128, 8), lambda i: (0, 0)),
            pl.BlockSpec((1, 1), lambda i: (0, 0)),
        ],
        out_specs=pl.BlockSpec((br, 8), lambda i: (i, 0)),
        out_shape=jax.ShapeDtypeStruct((r, 8), jnp.float32),
    )(partials, seg, b2)


def kernel(node_feats, node_xy, node_adj_ids, edge_ids, Wn1, bn1, Wn2, bn2,
           We1, be1, We2, be2):
    n = node_feats.shape[0]
    d = node_feats.shape[1]
    e = edge_ids.shape[1]

    npad = -(-n // (_NW * _RB)) * (_NW * _RB)
    nf = jnp.pad(node_feats, ((0, npad - n), (0, 0)))

    node_scores, t1, t2, amax = _tc_tables(
        nf, Wn1, bn1.reshape(1, -1), Wn2.reshape(1, -1),
        bn2.reshape(1, 1), We1[:d], We1[d:], be1.reshape(1, -1))

    p1, p2 = _sc_pack(t1, t2, amax)

    epad = -(-e // (_NW * _B)) * (_NW * _B)
    esrc = jnp.pad(edge_ids[0], (0, epad - e))
    edst = jnp.pad(edge_ids[1], (0, epad - e))

    partials = _sc_edge_partials(p1, p2, esrc, edst, We2.reshape(-1), amax)
    seg = (jnp.arange(128)[:, None] // _L
           == jnp.arange(8)[None, :]).astype(jnp.float32)
    sums = _tc_finalize(partials.reshape(epad * _L // 128, 128), seg,
                        be2.reshape(1, 1))
    edge_scores = sums.reshape(epad, 1)[:e]
    return (node_scores[:n], edge_scores)


# speedup vs baseline: 1.7154x; 1.0187x over previous
import functools

import jax
import jax.numpy as jnp
from jax import lax
from jax.experimental import pallas as pl
from jax.experimental.pallas import tpu as pltpu
from jax.experimental.pallas import tpu_sc as plsc

_NW = 32
_B = 128
_L = 16
_H = 128
_QMAX = 32000.0
_RB = 64

_SC_PARAMS = pltpu.CompilerParams(use_tc_tiling_on_sc=False)


def _tc_tables(nf, Wn1, bn1, Wn2, bn2, We1a, We1b, be1):
    n = nf.shape[0]
    d = nf.shape[1]
    bn = 512
    assert n % bn == 0

    def body(nf_ref, wn1_ref, bn1_ref, wn2_ref, bn2_ref, we1a_ref, we1b_ref,
             be1_ref, ns_ref, t1_ref, t2_ref, am_ref):
        i = pl.program_id(0)
        x = nf_ref[...]
        h = jnp.maximum(
            jnp.dot(x, wn1_ref[...], preferred_element_type=jnp.float32)
            + bn1_ref[...], 0.0)
        ns_ref[...] = (jnp.sum(h * wn2_ref[...], axis=1, keepdims=True)
                       + bn2_ref[...])
        t1 = (jnp.dot(x, we1a_ref[...], preferred_element_type=jnp.float32)
              + be1_ref[...])
        t2 = jnp.dot(x, we1b_ref[...], preferred_element_type=jnp.float32)
        t1_ref[...] = t1
        t2_ref[...] = t2
        bm = jnp.maximum(jnp.max(jnp.abs(t1)), jnp.max(jnp.abs(t2)))

        @pl.when(i == 0)
        def _():
            am_ref[...] = jnp.full((1, _H), 1e-30, jnp.float32)

        am_ref[...] = jnp.maximum(am_ref[...], bm)

    return pl.pallas_call(
        body,
        grid=(n // bn,),
        in_specs=[
            pl.BlockSpec((bn, d), lambda i: (i, 0)),
            pl.BlockSpec((d, _H), lambda i: (0, 0)),
            pl.BlockSpec((1, _H), lambda i: (0, 0)),
            pl.BlockSpec((1, _H), lambda i: (0, 0)),
            pl.BlockSpec((1, 1), lambda i: (0, 0)),
            pl.BlockSpec((d, _H), lambda i: (0, 0)),
            pl.BlockSpec((d, _H), lambda i: (0, 0)),
            pl.BlockSpec((1, _H), lambda i: (0, 0)),
        ],
        out_specs=[
            pl.BlockSpec((bn, 1), lambda i: (i, 0)),
            pl.BlockSpec((bn, _H), lambda i: (i, 0)),
            pl.BlockSpec((bn, _H), lambda i: (i, 0)),
            pl.BlockSpec((1, _H), lambda i: (0, 0)),
        ],
        out_shape=[
            jax.ShapeDtypeStruct((n, 1), jnp.float32),
            jax.ShapeDtypeStruct((n, _H), jnp.float32),
            jax.ShapeDtypeStruct((n, _H), jnp.float32),
            jax.ShapeDtypeStruct((1, _H), jnp.float32),
        ],
    )(nf, Wn1, bn1, Wn2, bn2, We1a, We1b, be1)


def _sc_pack(t1, t2, amax):
    n = t1.shape[0]
    rows_w = n // _NW
    assert rows_w % _RB == 0
    mesh = plsc.VectorSubcoreMesh(core_axis_name="c", subcore_axis_name="s")

    @functools.partial(
        pl.kernel,
        mesh=mesh,
        compiler_params=_SC_PARAMS,
        out_type=[
            jax.ShapeDtypeStruct((n, _H // 2), jnp.int32),
            jax.ShapeDtypeStruct((n, _H // 2), jnp.int32),
        ],
        scratch_types=[
            pltpu.VMEM((_RB, _H), jnp.float32),
            pltpu.VMEM((_RB, _H), jnp.float32),
            pltpu.VMEM((_RB, _H // 2), jnp.int32),
            pltpu.VMEM((_RB, _H // 2), jnp.int32),
            pltpu.VMEM((_H,), jnp.float32),
        ],
    )
    def k(t1_hbm, t2_hbm, am_hbm, p1_hbm, p2_hbm, t1v, t2v, p1v, p2v, amv):
        wid = lax.axis_index("s") * 2 + lax.axis_index("c")
        pltpu.sync_copy(am_hbm.at[0], amv)
        inv = _QMAX / amv[pl.ds(0, _L)]

        def chunk_body(c, carry):
            r0 = wid * rows_w + c * _RB
            pltpu.sync_copy(t1_hbm.at[pl.ds(r0, _RB)], t1v)
            pltpu.sync_copy(t2_hbm.at[pl.ds(r0, _RB)], t2v)

            @plsc.parallel_loop(0, _RB, 1, unroll=2)
            def row_body(r):
                for tv, pv in ((t1v, p1v), (t2v, p2v)):
                    for j in range(_H // (2 * _L)):
                        lo = tv[r, pl.ds(_L * j, _L)] * inv
                        hi = tv[r, pl.ds(_H // 2 + _L * j, _L)] * inv
                        qlo = (lo + jnp.where(lo >= 0, 0.5, -0.5)
                               ).astype(jnp.int32)
                        qhi = (hi + jnp.where(hi >= 0, 0.5, -0.5)
                               ).astype(jnp.int32)
                        pv[r, pl.ds(_L * j, _L)] = (
                            (qlo & 0xFFFF) | (qhi << 16))

            pltpu.sync_copy(p1v, p1_hbm.at[pl.ds(r0, _RB)])
            pltpu.sync_copy(p2v, p2_hbm.at[pl.ds(r0, _RB)])
            return carry

        lax.fori_loop(0, rows_w // _RB, chunk_body, 0)

    return k(t1, t2, amax)


def _sc_edge_partials(p1, p2, esrc, edst, w2, amax):
    epad = esrc.shape[0]
    nc = epad // (_NW * _B)
    assert nc % 2 == 0
    mesh = plsc.VectorSubcoreMesh(core_axis_name="c", subcore_axis_name="s")

    @functools.partial(
        pl.kernel,
        mesh=mesh,
        compiler_params=_SC_PARAMS,
        out_type=jax.ShapeDtypeStruct((epad * _L,), jnp.float32),
        scratch_types=[
            pltpu.VMEM((2, _B), jnp.int32),
            pltpu.VMEM((2, _B), jnp.int32),
            pltpu.VMEM((2, _B, _H // 2), jnp.int32),
            pltpu.VMEM((2, _B, _H // 2), jnp.int32),
            pltpu.VMEM((2, _B * _L), jnp.float32),
            pltpu.VMEM((_H,), jnp.float32),
            pltpu.VMEM((_H,), jnp.float32),
            pltpu.SemaphoreType.DMA,
            pltpu.SemaphoreType.DMA,
            pltpu.SemaphoreType.DMA,
            pltpu.SemaphoreType.DMA,
            pltpu.SemaphoreType.DMA,
            pltpu.SemaphoreType.DMA,
        ],
    )
    def k(p1_hbm, p2_hbm, esrc_hbm, edst_hbm, w2_hbm, am_hbm, out_hbm,
          sidx2, didx2, srows2, drows2, outv2, w2v, amv,
          sg0, sg1, si0, si1, so0, so1):
        sem_g, sem_i, sem_o = [sg0, sg1], [si0, si1], [so0, so1]
        wid = lax.axis_index("s") * 2 + lax.axis_index("c")
        pltpu.sync_copy(w2_hbm, w2v)
        pltpu.sync_copy(am_hbm.at[0], amv)
        step = amv[pl.ds(0, _L)] * (1.0 / _QMAX)
        w2c = [w2v[pl.ds(_L * j, _L)] * step for j in range(_H // _L)]
        nwc = _H // (2 * _L)

        def base_of(c):
            return (c * _NW + wid) * _B

        def issue_idx(c, b):
            base = base_of(c)
            pltpu.async_copy(esrc_hbm.at[pl.ds(base, _B)], sidx2.at[b],
                             sem_i[b])
            pltpu.async_copy(edst_hbm.at[pl.ds(base, _B)], didx2.at[b],
                             sem_i[b])

        def wait_idx(b):
            pltpu.make_async_copy(esrc_hbm.at[pl.ds(0, _B)], sidx2.at[b],
                                  sem_i[b]).wait()
            pltpu.make_async_copy(esrc_hbm.at[pl.ds(0, _B)], didx2.at[b],
                                  sem_i[b]).wait()

        def issue_gather(b):
            pltpu.async_copy(p1_hbm.at[sidx2.at[b]], srows2.at[b], sem_g[b])
            pltpu.async_copy(p2_hbm.at[didx2.at[b]], drows2.at[b], sem_g[b])

        def wait_gather(b):
            pltpu.make_async_copy(p1_hbm.at[sidx2.at[b]], srows2.at[b],
                                  sem_g[b]).wait()
            pltpu.make_async_copy(p2_hbm.at[didx2.at[b]], drows2.at[b],
                                  sem_g[b]).wait()

        def issue_out(c, b):
            pltpu.async_copy(outv2.at[b],
                             out_hbm.at[pl.ds(base_of(c) * _L, _B * _L)],
                             sem_o[b])

        def wait_out(b):
            pltpu.make_async_copy(outv2.at[b],
                                  out_hbm.at[pl.ds(0, _B * _L)],
                                  sem_o[b]).wait()

        def compute(b):
            srows, drows, outv = srows2.at[b], drows2.at[b], outv2.at[b]

            @plsc.parallel_loop(0, _B, 1, unroll=4)
            def edge_body(e):
                parts = []
                for j in range(nwc):
                    sl = pl.ds(_L * j, _L)
                    sw = srows[e, sl]
                    dw = drows[e, sl]
                    u_lo = ((sw << 16) >> 16) + ((dw << 16) >> 16)
                    u_hi = (sw >> 16) + (dw >> 16)
                    f_lo = jnp.maximum(u_lo, 0).astype(jnp.float32)
                    f_hi = jnp.maximum(u_hi, 0).astype(jnp.float32)
                    parts.append(f_lo * w2c[j])
                    parts.append(f_hi * w2c[j + nwc])
                while len(parts) > 1:
                    parts = [a + b_ for a, b_ in
                             zip(parts[::2], parts[1::2])]
                outv[pl.ds(e * _L, _L)] = parts[0]

        issue_idx(0, 0)
        wait_idx(0)
        issue_gather(0)
        issue_idx(1, 1)

        def pair_body(kk, carry):
            for b in (0, 1):
                c = 2 * kk + b
                wait_idx(b ^ 1)
                wait_gather(b)
                issue_gather(b ^ 1)
                issue_idx(jnp.minimum(c + 2, nc - 1), b)
                @pl.when(c >= 2)
                def _():
                    wait_out(b)
                compute(b)
                issue_out(c, b)
            return carry

        lax.fori_loop(0, nc // 2, pair_body, 0)
        wait_gather(0)
        wait_idx(1)
        wait_out(0)
        wait_out(1)

    return k(p1, p2, esrc, edst, w2, amax)


def _tc_finalize(partials, seg, b2):
    r = partials.shape[0]
    br = 512
    assert r % br == 0

    def body(p_ref, s_ref, b2_ref, o_ref):
        o_ref[...] = (jnp.dot(p_ref[...], s_ref[...],
                              preferred_element_type=jnp.float32)
                      + b2_ref[...])

    return pl.pallas_call(
        body,
        grid=(r // br,),
        in_specs=[
            pl.BlockSpec((br, 128), lambda i: (i, 0)),
            pl.BlockSpec((128, 8), lambda i: (0, 0)),
            pl.BlockSpec((1, 1), lambda i: (0, 0)),
        ],
        out_specs=pl.BlockSpec((br, 8), lambda i: (i, 0)),
        out_shape=jax.ShapeDtypeStruct((r, 8), jnp.float32),
    )(partials, seg, b2)


def kernel(node_feats, node_xy, node_adj_ids, edge_ids, Wn1, bn1, Wn2, bn2,
           We1, be1, We2, be2):
    n = node_feats.shape[0]
    d = node_feats.shape[1]
    e = edge_ids.shape[1]

    npad = -(-n // (_NW * _RB)) * (_NW * _RB)
    nf = jnp.pad(node_feats, ((0, npad - n), (0, 0)))

    node_scores, t1, t2, amax = _tc_tables(
        nf, Wn1, bn1.reshape(1, -1), Wn2.reshape(1, -1),
        bn2.reshape(1, 1), We1[:d], We1[d:], be1.reshape(1, -1))

    p1, p2 = _sc_pack(t1, t2, amax)

    epad = -(-e // (2 * _NW * _B)) * (2 * _NW * _B)
    esrc = jnp.pad(edge_ids[0], (0, epad - e))
    edst = jnp.pad(edge_ids[1], (0, epad - e))

    partials = _sc_edge_partials(p1, p2, esrc, edst, We2.reshape(-1), amax)
    seg = (jnp.arange(128)[:, None] // _L
           == jnp.arange(8)[None, :]).astype(jnp.float32)
    sums = _tc_finalize(partials.reshape(epad * _L // 128, 128), seg,
                        be2.reshape(1, 1))
    edge_scores = sums.reshape(epad, 1)[:e]
    return (node_scores[:n], edge_scores)
